# baseline (speedup 1.0000x reference)
import functools

import jax
import jax.numpy as jnp
from jax import lax
from jax.experimental import pallas as pl
from jax.experimental.pallas import tpu as pltpu
from jax.experimental.pallas import tpu_sc as plsc

N = 100000
E = 1600000
F_IN = 5
H = 64
L = 6
EPS = 1e-5

NC, NS = 2, 16
NW = NC * NS
ROW = 128
N_PAD = 100352
NB = N_PAD // 1024
EPW = 392 * ROW
E_PAD = NW * EPW
EB = E_PAD // 1024
PIECES = 7
RPP = 392 // PIECES
STRIPE = N_PAD // NS
FC = 16
NCH = H // FC
KG = 14
KE = 4

_mesh = plsc.VectorSubcoreMesh(
    core_axis_name="c", subcore_axis_name="s", num_cores=NC, num_subcores=NS)


def _zero_vmem_2d(ref, nrows):
    def bd(k, _):
        ref[k] = jnp.zeros((FC,), jnp.float32)
        return 0
    lax.fori_loop(0, nrows, bd, 0)


@functools.partial(
    pl.kernel,
    out_type=jax.ShapeDtypeStruct((NC, N_PAD), jnp.float32),
    mesh=_mesh,
    compiler_params=pltpu.CompilerParams(use_tc_tiling_on_sc=False),
    scratch_types=[
        pltpu.VMEM((RPP, ROW), jnp.int32),
        pltpu.VMEM((ROW,), jnp.float32),
        pltpu.VMEM((STRIPE,), jnp.float32),
        pltpu.VMEM_SHARED((N_PAD,), jnp.float32),
        pltpu.SemaphoreType.DMA,
    ],
)
def _deg_sc(dst2d, degp, didx, ones, zbuf, acc, sem):
    c = lax.axis_index("c")
    s = lax.axis_index("s")
    w = c * NS + s

    def zb(k, _):
        zbuf[pl.ds(k * 16, 16)] = jnp.zeros((16,), jnp.float32)
        return 0
    lax.fori_loop(0, STRIPE // 16, zb, 0)
    for k in range(ROW // 16):
        ones[pl.ds(k * 16, 16)] = jnp.ones((16,), jnp.float32)
    pltpu.sync_copy(zbuf, acc.at[pl.ds(s * STRIPE, STRIPE)])
    plsc.subcore_barrier()

    for p in range(PIECES):
        pltpu.sync_copy(dst2d.at[pl.ds(w * 392 + p * RPP, RPP)], didx)

        def body(g, _):
            ds = [pltpu.async_copy(ones, acc.at[didx.at[g * KG + b]], sem,
                                   add=True) for b in range(KG)]
            for d in ds:
                d.wait()
            return 0
        lax.fori_loop(0, RPP // KG, body, 0)
    plsc.subcore_barrier()
    pltpu.sync_copy(acc.at[pl.ds(s * STRIPE, STRIPE)],
                    degp.at[c, pl.ds(s * STRIPE, STRIPE)])


@functools.partial(
    pl.kernel,
    out_type=jax.ShapeDtypeStruct((NC, NCH, N_PAD, FC), jnp.float32),
    mesh=_mesh,
    compiler_params=pltpu.CompilerParams(use_tc_tiling_on_sc=False),
    scratch_types=[
        pltpu.VMEM((KG, ROW), jnp.int32),
        pltpu.VMEM((KG, ROW, FC), jnp.float32),
        pltpu.VMEM_SHARED((N_PAD, FC), jnp.float32),
        pltpu.SemaphoreType.DMA,
        pltpu.SemaphoreType.DMA,
    ],
)
def _layer_sc(src2d, dst2d, hs0, hs1, hs2, hs3, out, idxb, ring,
              acc, semg, semsc):
    c = lax.axis_index("c")
    s = lax.axis_index("s")
    w = c * NS + s

    hs_chunks = (hs0, hs1, hs2, hs3)

    for ch in range(NCH):
        def zr(k, _):
            ring[0, k] = jnp.zeros((FC,), jnp.float32)
            return 0
        lax.fori_loop(0, ROW, zr, 0)

        def zc(i, _):
            dz = [pltpu.async_copy(
                ring.at[0],
                acc.at[pl.ds(s * STRIPE + (i * 7 + t) * ROW, ROW)], semsc)
                for t in range(7)]
            for d in dz:
                d.wait()
            return 0
        lax.fori_loop(0, STRIPE // ROW // 7, zc, 0)
        plsc.subcore_barrier()

        def body(g, _):
            tab = hs_chunks[ch]
            base = w * 392 + g * KG
            pltpu.sync_copy(src2d.at[pl.ds(base, KG)], idxb)
            dg = [pltpu.async_copy(tab.at[idxb.at[b]],
                                   ring.at[b], semg) for b in range(KG)]
            for d in dg:
                d.wait()
            pltpu.sync_copy(dst2d.at[pl.ds(base, KG)], idxb)
            dsc = [pltpu.async_copy(ring.at[b], acc.at[idxb.at[b]],
                                    semsc, add=True) for b in range(KG)]
            for d in dsc:
                d.wait()
            return 0
        lax.fori_loop(0, 392 // KG, body, 0)
        plsc.subcore_barrier()
        pltpu.sync_copy(
            acc.at[pl.ds(s * STRIPE, STRIPE)],
            out.at[c, ch, pl.ds(s * STRIPE, STRIPE)])


@functools.partial(
    pl.kernel,
    out_type=(jax.ShapeDtypeStruct((E_PAD, H), jnp.float32),
              jax.ShapeDtypeStruct((E_PAD, H), jnp.float32)),
    mesh=_mesh,
    compiler_params=pltpu.CompilerParams(use_tc_tiling_on_sc=False),
    scratch_types=[
        pltpu.VMEM((RPP, ROW), jnp.int32),
        pltpu.VMEM((RPP, ROW), jnp.int32),
        pltpu.VMEM((KE, ROW, H), jnp.float32),
        pltpu.VMEM((KE, ROW, H), jnp.float32),
        pltpu.SemaphoreType.DMA,
        pltpu.SemaphoreType.DMA,
        pltpu.SemaphoreType.DMA,
    ],
)
def _edge_sc(a_tab, b_tab, src2d, dst2d, u_out, v_out, sidx, didx, ringa,
             ringb, sema, semb, semw):
    c = lax.axis_index("c")
    s = lax.axis_index("s")
    w = c * NS + s
    for p in range(PIECES):
        pltpu.sync_copy(src2d.at[pl.ds(w * 392 + p * RPP, RPP)], sidx)
        pltpu.sync_copy(dst2d.at[pl.ds(w * 392 + p * RPP, RPP)], didx)

        def body(g, _):
            da = [pltpu.async_copy(a_tab.at[sidx.at[g * KE + b]],
                                   ringa.at[b], sema) for b in range(KE)]
            db = [pltpu.async_copy(b_tab.at[didx.at[g * KE + b]],
                                   ringb.at[b], semb) for b in range(KE)]
            for d in da + db:
                d.wait()
            dw = []
            for b in range(KE):
                base = (w * 392 + p * RPP + g * KE + b) * ROW
                dw.append(pltpu.async_copy(ringa.at[b],
                                           u_out.at[pl.ds(base, ROW)], semw))
                dw.append(pltpu.async_copy(ringb.at[b],
                                           v_out.at[pl.ds(base, ROW)], semw))
            for d in dw:
                d.wait()
            return 0
        lax.fori_loop(0, RPP // KE, body, 0)


def _dinv_body(p_ref, o_ref):
    psum = p_ref[0] + p_ref[1] + 1.0
    r = lax.broadcasted_iota(jnp.int32, (N_PAD // 128, 128), 0)
    l2 = lax.broadcasted_iota(jnp.int32, (N_PAD // 128, 128), 1)
    flat = r * 128 + l2
    o_ref[...] = jnp.where(flat < N, lax.rsqrt(psum), 0.0)


def _dinv_tc(degp):
    return pl.pallas_call(
        _dinv_body,
        out_shape=jax.ShapeDtypeStruct((N_PAD // 128, 128), jnp.float32),
    )(degp.reshape(NC, N_PAD // 128, 128))


def _input_body(x_ref, win_ref, bin_ref, w0_ref, dinv_ref, h_ref, hsf_ref,
                hs0_ref, hs1_ref, hs2_ref, hs3_ref):
    h = jnp.maximum(
        jnp.dot(x_ref[...], win_ref[...],
                preferred_element_type=jnp.float32) + bin_ref[...], 0.0)
    hs = jnp.dot(h, w0_ref[...], preferred_element_type=jnp.float32) \
        * dinv_ref[...]
    h_ref[...] = h
    hsf_ref[...] = hs
    hs0_ref[...] = hs[:, 0:16]
    hs1_ref[...] = hs[:, 16:32]
    hs2_ref[...] = hs[:, 32:48]
    hs3_ref[...] = hs[:, 48:64]


def _input_tc(x8, w_in8, b_in, w0, dinv):
    bs = 1024
    return pl.pallas_call(
        _input_body,
        grid=(NB,),
        in_specs=[
            pl.BlockSpec((bs, 8), lambda i: (i, 0)),
            pl.BlockSpec((8, H), lambda i: (0, 0)),
            pl.BlockSpec((1, H), lambda i: (0, 0)),
            pl.BlockSpec((H, H), lambda i: (0, 0)),
            pl.BlockSpec((bs, 1), lambda i: (i, 0)),
        ],
        out_specs=[
            pl.BlockSpec((bs, H), lambda i: (i, 0)),
            pl.BlockSpec((bs, H), lambda i: (i, 0)),
            pl.BlockSpec((bs, FC), lambda i: (i, 0)),
            pl.BlockSpec((bs, FC), lambda i: (i, 0)),
            pl.BlockSpec((bs, FC), lambda i: (i, 0)),
            pl.BlockSpec((bs, FC), lambda i: (i, 0)),
        ],
        out_shape=[
            jax.ShapeDtypeStruct((N_PAD, H), jnp.float32),
            jax.ShapeDtypeStruct((N_PAD, H), jnp.float32),
            jax.ShapeDtypeStruct((N_PAD, FC), jnp.float32),
            jax.ShapeDtypeStruct((N_PAD, FC), jnp.float32),
            jax.ShapeDtypeStruct((N_PAD, FC), jnp.float32),
            jax.ShapeDtypeStruct((N_PAD, FC), jnp.float32),
        ],
    )(x8, w_in8, b_in, w0, dinv)


def _stats_body(p_ref, hsf_ref, dinv_ref, b_ref, hn_ref, st_ref, acc_ref):
    i = pl.program_id(0)
    psum = p_ref[0] + p_ref[1]
    pcat = jnp.concatenate([psum[q] for q in range(NCH)], axis=-1)
    hn = dinv_ref[...] * (pcat + hsf_ref[...]) + b_ref[...]
    hn_ref[...] = hn
    r = lax.broadcasted_iota(jnp.int32, (1024, 1), 0) + i * 1024
    mask = jnp.where(r < N, 1.0, 0.0)
    hm = hn * mask
    ps = jnp.sum(hm, axis=0, keepdims=True)
    ps2 = jnp.sum(hm * hn, axis=0, keepdims=True)

    @pl.when(i == 0)
    def _():
        acc_ref[...] = jnp.zeros_like(acc_ref)

    acc_ref[0:1] += ps
    acc_ref[1:2] += ps2

    @pl.when(i == NB - 1)
    def _():
        st_ref[...] = acc_ref[...]


def _stats_tc(partials, hsf, dinv, b):
    bs = 1024
    return pl.pallas_call(
        _stats_body,
        grid=(NB,),
        in_specs=[
            pl.BlockSpec((NC, NCH, bs, FC), lambda i: (0, 0, i, 0)),
            pl.BlockSpec((bs, H), lambda i: (i, 0)),
            pl.BlockSpec((bs, 1), lambda i: (i, 0)),
            pl.BlockSpec((1, H), lambda i: (0, 0)),
        ],
        out_specs=[
            pl.BlockSpec((bs, H), lambda i: (i, 0)),
            pl.BlockSpec((8, H), lambda i: (0, 0)),
        ],
        out_shape=[
            jax.ShapeDtypeStruct((N_PAD, H), jnp.float32),
            jax.ShapeDtypeStruct((8, H), jnp.float32),
        ],
        scratch_shapes=[pltpu.VMEM((8, H), jnp.float32)],
    )(partials, hsf, dinv, b)


def _apply_body(hn_ref, st_ref, g_ref, be_ref, h_ref, dinv_ref, wn_ref,
                ho_ref, hsf_ref, hs0_ref, hs1_ref, hs2_ref, hs3_ref):
    m = st_ref[0:1] * (1.0 / N)
    ex2 = st_ref[1:2] * (1.0 / N)
    var = ex2 - m * m
    inv = lax.rsqrt(var + EPS)
    hn = (hn_ref[...] - m) * inv * g_ref[...] + be_ref[...]
    h_new = h_ref[...] + jnp.maximum(hn, 0.0)
    hs = jnp.dot(h_new, wn_ref[...], preferred_element_type=jnp.float32) \
        * dinv_ref[...]
    ho_ref[...] = h_new
    hsf_ref[...] = hs
    hs0_ref[...] = hs[:, 0:16]
    hs1_ref[...] = hs[:, 16:32]
    hs2_ref[...] = hs[:, 32:48]
    hs3_ref[...] = hs[:, 48:64]


def _apply_tc(hn, stats, gamma, beta, h, dinv, w_next):
    bs = 1024
    return pl.pallas_call(
        _apply_body,
        grid=(NB,),
        in_specs=[
            pl.BlockSpec((bs, H), lambda i: (i, 0)),
            pl.BlockSpec((8, H), lambda i: (0, 0)),
            pl.BlockSpec((1, H), lambda i: (0, 0)),
            pl.BlockSpec((1, H), lambda i: (0, 0)),
            pl.BlockSpec((bs, H), lambda i: (i, 0)),
            pl.BlockSpec((bs, 1), lambda i: (i, 0)),
            pl.BlockSpec((H, H), lambda i: (0, 0)),
        ],
        out_specs=[
            pl.BlockSpec((bs, H), lambda i: (i, 0)),
            pl.BlockSpec((bs, H), lambda i: (i, 0)),
            pl.BlockSpec((bs, FC), lambda i: (i, 0)),
            pl.BlockSpec((bs, FC), lambda i: (i, 0)),
            pl.BlockSpec((bs, FC), lambda i: (i, 0)),
            pl.BlockSpec((bs, FC), lambda i: (i, 0)),
        ],
        out_shape=[
            jax.ShapeDtypeStruct((N_PAD, H), jnp.float32),
            jax.ShapeDtypeStruct((N_PAD, H), jnp.float32),
            jax.ShapeDtypeStruct((N_PAD, FC), jnp.float32),
            jax.ShapeDtypeStruct((N_PAD, FC), jnp.float32),
            jax.ShapeDtypeStruct((N_PAD, FC), jnp.float32),
            jax.ShapeDtypeStruct((N_PAD, FC), jnp.float32),
        ],
    )(hn, stats, gamma, beta, h, dinv, w_next)


def _final_body(hn_ref, st_ref, g_ref, be_ref, h_ref, wa_ref, wb_ref, ba_ref,
                a_ref, b_ref):
    m = st_ref[0:1] * (1.0 / N)
    ex2 = st_ref[1:2] * (1.0 / N)
    var = ex2 - m * m
    inv = lax.rsqrt(var + EPS)
    hn = (hn_ref[...] - m) * inv * g_ref[...] + be_ref[...]
    h_new = h_ref[...] + jnp.maximum(hn, 0.0)
    a_ref[...] = jnp.dot(h_new, wa_ref[...],
                         preferred_element_type=jnp.float32) + ba_ref[...]
    b_ref[...] = jnp.dot(h_new, wb_ref[...],
                         preferred_element_type=jnp.float32)


def _final_tc(hn, stats, gamma, beta, h, w_e1a, w_e1b, b_e1):
    bs = 1024
    return pl.pallas_call(
        _final_body,
        grid=(NB,),
        in_specs=[
            pl.BlockSpec((bs, H), lambda i: (i, 0)),
            pl.BlockSpec((8, H), lambda i: (0, 0)),
            pl.BlockSpec((1, H), lambda i: (0, 0)),
            pl.BlockSpec((1, H), lambda i: (0, 0)),
            pl.BlockSpec((bs, H), lambda i: (i, 0)),
            pl.BlockSpec((H, H), lambda i: (0, 0)),
            pl.BlockSpec((H, H), lambda i: (0, 0)),
            pl.BlockSpec((1, H), lambda i: (0, 0)),
        ],
        out_specs=[
            pl.BlockSpec((bs, H), lambda i: (i, 0)),
            pl.BlockSpec((bs, H), lambda i: (i, 0)),
        ],
        out_shape=[
            jax.ShapeDtypeStruct((N_PAD, H), jnp.float32),
            jax.ShapeDtypeStruct((N_PAD, H), jnp.float32),
        ],
    )(hn, stats, gamma, beta, h, w_e1a, w_e1b, b_e1)


def _edge_body(u_ref, v_ref, w2_ref, b2_ref, w3_ref, o_ref):
    z1 = jnp.maximum(u_ref[...] + v_ref[...], 0.0)
    z2 = jnp.maximum(
        jnp.dot(z1, w2_ref[...], preferred_element_type=jnp.float32)
        + b2_ref[...], 0.0)
    o_ref[...] = jnp.dot(z2, w3_ref[...], preferred_element_type=jnp.float32)


def _edge_tc(u, v, w2, b2, w3):
    bs = 1024
    return pl.pallas_call(
        _edge_body,
        grid=(EB,),
        in_specs=[
            pl.BlockSpec((bs, H), lambda i: (i, 0)),
            pl.BlockSpec((bs, H), lambda i: (i, 0)),
            pl.BlockSpec((H, 32), lambda i: (0, 0)),
            pl.BlockSpec((1, 32), lambda i: (0, 0)),
            pl.BlockSpec((32, 1), lambda i: (0, 0)),
        ],
        out_specs=pl.BlockSpec((bs, 1), lambda i: (i, 0)),
        out_shape=jax.ShapeDtypeStruct((E_PAD, 1), jnp.float32),
    )(u, v, w2, b2, w3)


def kernel(x, edge_index, params):
    src = edge_index[0]
    dst = edge_index[1]
    src_p = jnp.concatenate(
        [src, jnp.zeros((E_PAD - E,), jnp.int32)]).reshape(E_PAD // ROW, ROW)
    dst_p = jnp.concatenate(
        [dst, jnp.full((E_PAD - E,), N, jnp.int32)]).reshape(E_PAD // ROW, ROW)
    x8 = jnp.pad(x, ((0, N_PAD - N), (0, 8 - F_IN)))

    degp = _deg_sc(dst_p)
    dinv = _dinv_tc(degp).reshape(N_PAD, 1)

    b_in = params['b_in'].reshape(1, H)
    h, hsf, hs0, hs1, hs2, hs3 = _input_tc(
        x8, jnp.pad(params['W_in'], ((0, 8 - F_IN), (0, 0))), b_in,
        params['W_0'], dinv)

    for l in range(L):
        partials = _layer_sc(src_p, dst_p, hs0, hs1, hs2, hs3)
        hn, stats = _stats_tc(partials, hsf, dinv,
                              params[f'b_{l}'].reshape(1, H))
        gamma = params[f'gamma_{l}'].reshape(1, H)
        beta = params[f'beta_{l}'].reshape(1, H)
        if l < L - 1:
            h, hsf, hs0, hs1, hs2, hs3 = _apply_tc(
                hn, stats, gamma, beta, h, dinv, params[f'W_{l + 1}'])
        else:
            a_tab, b_tab = _final_tc(
                hn, stats, gamma, beta, h,
                params['W_e1'][:H], params['W_e1'][H:],
                params['b_e1'].reshape(1, H))

    u, v = _edge_sc(a_tab, b_tab, src_p, dst_p)
    logits = _edge_tc(u, v, params['W_e2'], params['b_e2'].reshape(1, 32),
                      params['W_e3'])
    return (logits.reshape(E_PAD) + params['b_e3'][0])[:E]


# speedup vs baseline: 1.0823x; 1.0823x over previous
import functools

import jax
import jax.numpy as jnp
from jax import lax
from jax.experimental import pallas as pl
from jax.experimental.pallas import tpu as pltpu
from jax.experimental.pallas import tpu_sc as plsc

N = 100000
E = 1600000
F_IN = 5
H = 64
L = 6
EPS = 1e-5

NC, NS = 2, 16
NW = NC * NS
ROW = 128
N_PAD = 100352
NB = N_PAD // 1024
EPW = 392 * ROW
E_PAD = NW * EPW
EB = E_PAD // 1024
PIECES = 7
RPP = 392 // PIECES
STRIPE = N_PAD // NS
FC = 16
NCH = H // FC
KG = 14
KE = 4

_mesh = plsc.VectorSubcoreMesh(
    core_axis_name="c", subcore_axis_name="s", num_cores=NC, num_subcores=NS)


def _zero_vmem_2d(ref, nrows):
    def bd(k, _):
        ref[k] = jnp.zeros((FC,), jnp.float32)
        return 0
    lax.fori_loop(0, nrows, bd, 0)


@functools.partial(
    pl.kernel,
    out_type=jax.ShapeDtypeStruct((NC, N_PAD), jnp.float32),
    mesh=_mesh,
    compiler_params=pltpu.CompilerParams(use_tc_tiling_on_sc=False),
    scratch_types=[
        pltpu.VMEM((RPP, ROW), jnp.int32),
        pltpu.VMEM((ROW,), jnp.float32),
        pltpu.VMEM((STRIPE,), jnp.float32),
        pltpu.VMEM_SHARED((N_PAD,), jnp.float32),
        pltpu.SemaphoreType.DMA,
    ],
)
def _deg_sc(dst2d, degp, didx, ones, zbuf, acc, sem):
    c = lax.axis_index("c")
    s = lax.axis_index("s")
    w = c * NS + s

    def zb(k, _):
        zbuf[pl.ds(k * 16, 16)] = jnp.zeros((16,), jnp.float32)
        return 0
    lax.fori_loop(0, STRIPE // 16, zb, 0)
    for k in range(ROW // 16):
        ones[pl.ds(k * 16, 16)] = jnp.ones((16,), jnp.float32)
    pltpu.sync_copy(zbuf, acc.at[pl.ds(s * STRIPE, STRIPE)])
    plsc.subcore_barrier()

    for p in range(PIECES):
        pltpu.sync_copy(dst2d.at[pl.ds(w * 392 + p * RPP, RPP)], didx)

        def body(g, _):
            ds = [pltpu.async_copy(ones, acc.at[didx.at[g * KG + b]], sem,
                                   add=True) for b in range(KG)]
            for d in ds:
                d.wait()
            return 0
        lax.fori_loop(0, RPP // KG, body, 0)
    plsc.subcore_barrier()
    pltpu.sync_copy(acc.at[pl.ds(s * STRIPE, STRIPE)],
                    degp.at[c, pl.ds(s * STRIPE, STRIPE)])


@functools.partial(
    pl.kernel,
    out_type=jax.ShapeDtypeStruct((NC, NCH, N_PAD, FC), jnp.float32),
    mesh=_mesh,
    compiler_params=pltpu.CompilerParams(use_tc_tiling_on_sc=False),
    scratch_types=[
        pltpu.VMEM((KG, ROW), jnp.int32),
        pltpu.VMEM((KG, ROW, FC), jnp.float32),
        pltpu.VMEM_SHARED((N_PAD, FC), jnp.float32),
        pltpu.SemaphoreType.DMA,
        pltpu.SemaphoreType.DMA,
    ],
)
def _layer_sc(srcq0, srcq1, srcq2, srcq3, dst2d, hs4, out, idxb, ring,
              acc, semg, semsc):
    c = lax.axis_index("c")
    s = lax.axis_index("s")
    w = c * NS + s

    src_tabs = (srcq0, srcq1, srcq2, srcq3)
    for ch in range(NCH):
        def zr(k, _):
            ring[0, k] = jnp.zeros((FC,), jnp.float32)
            return 0
        lax.fori_loop(0, ROW, zr, 0)

        def zc(i, _):
            dz = [pltpu.async_copy(
                ring.at[0],
                acc.at[pl.ds(s * STRIPE + (i * 7 + t) * ROW, ROW)], semsc)
                for t in range(7)]
            for d in dz:
                d.wait()
            return 0
        lax.fori_loop(0, STRIPE // ROW // 7, zc, 0)
        plsc.subcore_barrier()

        def body(g, _):
            base = w * 392 + g * KG
            pltpu.sync_copy(src_tabs[ch].at[pl.ds(base, KG)], idxb)
            dg = [pltpu.async_copy(hs4.at[idxb.at[b]],
                                   ring.at[b], semg) for b in range(KG)]
            for d in dg:
                d.wait()
            pltpu.sync_copy(dst2d.at[pl.ds(base, KG)], idxb)
            dsc = [pltpu.async_copy(ring.at[b], acc.at[idxb.at[b]],
                                    semsc, add=True) for b in range(KG)]
            for d in dsc:
                d.wait()
            return 0
        lax.fori_loop(0, 392 // KG, body, 0)
        plsc.subcore_barrier()
        pltpu.sync_copy(
            acc.at[pl.ds(s * STRIPE, STRIPE)],
            out.at[c, ch, pl.ds(s * STRIPE, STRIPE)])


@functools.partial(
    pl.kernel,
    out_type=(jax.ShapeDtypeStruct((E_PAD, H), jnp.float32),
              jax.ShapeDtypeStruct((E_PAD, H), jnp.float32)),
    mesh=_mesh,
    compiler_params=pltpu.CompilerParams(use_tc_tiling_on_sc=False),
    scratch_types=[
        pltpu.VMEM((RPP, ROW), jnp.int32),
        pltpu.VMEM((RPP, ROW), jnp.int32),
        pltpu.VMEM((KE, ROW, H), jnp.float32),
        pltpu.VMEM((KE, ROW, H), jnp.float32),
        pltpu.SemaphoreType.DMA,
        pltpu.SemaphoreType.DMA,
        pltpu.SemaphoreType.DMA,
    ],
)
def _edge_sc(a_tab, b_tab, src2d, dst2d, u_out, v_out, sidx, didx, ringa,
             ringb, sema, semb, semw):
    c = lax.axis_index("c")
    s = lax.axis_index("s")
    w = c * NS + s
    for p in range(PIECES):
        pltpu.sync_copy(src2d.at[pl.ds(w * 392 + p * RPP, RPP)], sidx)
        pltpu.sync_copy(dst2d.at[pl.ds(w * 392 + p * RPP, RPP)], didx)

        def body(g, _):
            da = [pltpu.async_copy(a_tab.at[sidx.at[g * KE + b]],
                                   ringa.at[b], sema) for b in range(KE)]
            db = [pltpu.async_copy(b_tab.at[didx.at[g * KE + b]],
                                   ringb.at[b], semb) for b in range(KE)]
            for d in da + db:
                d.wait()
            dw = []
            for b in range(KE):
                base = (w * 392 + p * RPP + g * KE + b) * ROW
                dw.append(pltpu.async_copy(ringa.at[b],
                                           u_out.at[pl.ds(base, ROW)], semw))
                dw.append(pltpu.async_copy(ringb.at[b],
                                           v_out.at[pl.ds(base, ROW)], semw))
            for d in dw:
                d.wait()
            return 0
        lax.fori_loop(0, RPP // KE, body, 0)


def _dinv_body(p_ref, o_ref):
    psum = p_ref[0] + p_ref[1] + 1.0
    r = lax.broadcasted_iota(jnp.int32, (N_PAD // 128, 128), 0)
    l2 = lax.broadcasted_iota(jnp.int32, (N_PAD // 128, 128), 1)
    flat = r * 128 + l2
    o_ref[...] = jnp.where(flat < N, lax.rsqrt(psum), 0.0)


def _dinv_tc(degp):
    return pl.pallas_call(
        _dinv_body,
        out_shape=jax.ShapeDtypeStruct((N_PAD // 128, 128), jnp.float32),
    )(degp.reshape(NC, N_PAD // 128, 128))


def _input_body(x_ref, win_ref, bin_ref, w0_ref, dinv_ref, h_ref, hsf_ref):
    h = jnp.maximum(
        jnp.dot(x_ref[...], win_ref[...],
                preferred_element_type=jnp.float32) + bin_ref[...], 0.0)
    hs = jnp.dot(h, w0_ref[...], preferred_element_type=jnp.float32) \
        * dinv_ref[...]
    h_ref[...] = h
    hsf_ref[...] = hs


def _input_tc(x8, w_in8, b_in, w0, dinv):
    bs = 1024
    return pl.pallas_call(
        _input_body,
        grid=(NB,),
        in_specs=[
            pl.BlockSpec((bs, 8), lambda i: (i, 0)),
            pl.BlockSpec((8, H), lambda i: (0, 0)),
            pl.BlockSpec((1, H), lambda i: (0, 0)),
            pl.BlockSpec((H, H), lambda i: (0, 0)),
            pl.BlockSpec((bs, 1), lambda i: (i, 0)),
        ],
        out_specs=[
            pl.BlockSpec((bs, H), lambda i: (i, 0)),
            pl.BlockSpec((bs, H), lambda i: (i, 0)),
        ],
        out_shape=[
            jax.ShapeDtypeStruct((N_PAD, H), jnp.float32),
            jax.ShapeDtypeStruct((N_PAD, H), jnp.float32),
        ],
    )(x8, w_in8, b_in, w0, dinv)


def _stats_body(p_ref, hsf_ref, dinv_ref, b_ref, hn_ref, st_ref, acc_ref):
    i = pl.program_id(0)
    psum = p_ref[0] + p_ref[1]
    pcat = jnp.concatenate([psum[q] for q in range(NCH)], axis=-1)
    hn = dinv_ref[...] * (pcat + hsf_ref[...]) + b_ref[...]
    hn_ref[...] = hn
    r = lax.broadcasted_iota(jnp.int32, (1024, 1), 0) + i * 1024
    mask = jnp.where(r < N, 1.0, 0.0)
    hm = hn * mask
    ps = jnp.sum(hm, axis=0, keepdims=True)
    ps2 = jnp.sum(hm * hn, axis=0, keepdims=True)

    @pl.when(i == 0)
    def _():
        acc_ref[...] = jnp.zeros_like(acc_ref)

    acc_ref[0:1] += ps
    acc_ref[1:2] += ps2

    @pl.when(i == NB - 1)
    def _():
        st_ref[...] = acc_ref[...]


def _stats_tc(partials, hsf, dinv, b):
    bs = 1024
    return pl.pallas_call(
        _stats_body,
        grid=(NB,),
        in_specs=[
            pl.BlockSpec((NC, NCH, bs, FC), lambda i: (0, 0, i, 0)),
            pl.BlockSpec((bs, H), lambda i: (i, 0)),
            pl.BlockSpec((bs, 1), lambda i: (i, 0)),
            pl.BlockSpec((1, H), lambda i: (0, 0)),
        ],
        out_specs=[
            pl.BlockSpec((bs, H), lambda i: (i, 0)),
            pl.BlockSpec((8, H), lambda i: (0, 0)),
        ],
        out_shape=[
            jax.ShapeDtypeStruct((N_PAD, H), jnp.float32),
            jax.ShapeDtypeStruct((8, H), jnp.float32),
        ],
        scratch_shapes=[pltpu.VMEM((8, H), jnp.float32)],
    )(partials, hsf, dinv, b)


def _apply_body(hn_ref, st_ref, g_ref, be_ref, h_ref, dinv_ref, wn_ref,
                ho_ref, hsf_ref):
    m = st_ref[0:1] * (1.0 / N)
    ex2 = st_ref[1:2] * (1.0 / N)
    var = ex2 - m * m
    inv = lax.rsqrt(var + EPS)
    hn = (hn_ref[...] - m) * inv * g_ref[...] + be_ref[...]
    h_new = h_ref[...] + jnp.maximum(hn, 0.0)
    hs = jnp.dot(h_new, wn_ref[...], preferred_element_type=jnp.float32) \
        * dinv_ref[...]
    ho_ref[...] = h_new
    hsf_ref[...] = hs


def _apply_tc(hn, stats, gamma, beta, h, dinv, w_next):
    bs = 1024
    return pl.pallas_call(
        _apply_body,
        grid=(NB,),
        in_specs=[
            pl.BlockSpec((bs, H), lambda i: (i, 0)),
            pl.BlockSpec((8, H), lambda i: (0, 0)),
            pl.BlockSpec((1, H), lambda i: (0, 0)),
            pl.BlockSpec((1, H), lambda i: (0, 0)),
            pl.BlockSpec((bs, H), lambda i: (i, 0)),
            pl.BlockSpec((bs, 1), lambda i: (i, 0)),
            pl.BlockSpec((H, H), lambda i: (0, 0)),
        ],
        out_specs=[
            pl.BlockSpec((bs, H), lambda i: (i, 0)),
            pl.BlockSpec((bs, H), lambda i: (i, 0)),
        ],
        out_shape=[
            jax.ShapeDtypeStruct((N_PAD, H), jnp.float32),
            jax.ShapeDtypeStruct((N_PAD, H), jnp.float32),
        ],
    )(hn, stats, gamma, beta, h, dinv, w_next)


def _final_body(hn_ref, st_ref, g_ref, be_ref, h_ref, wa_ref, wb_ref, ba_ref,
                a_ref, b_ref):
    m = st_ref[0:1] * (1.0 / N)
    ex2 = st_ref[1:2] * (1.0 / N)
    var = ex2 - m * m
    inv = lax.rsqrt(var + EPS)
    hn = (hn_ref[...] - m) * inv * g_ref[...] + be_ref[...]
    h_new = h_ref[...] + jnp.maximum(hn, 0.0)
    a_ref[...] = jnp.dot(h_new, wa_ref[...],
                         preferred_element_type=jnp.float32) + ba_ref[...]
    b_ref[...] = jnp.dot(h_new, wb_ref[...],
                         preferred_element_type=jnp.float32)


def _final_tc(hn, stats, gamma, beta, h, w_e1a, w_e1b, b_e1):
    bs = 1024
    return pl.pallas_call(
        _final_body,
        grid=(NB,),
        in_specs=[
            pl.BlockSpec((bs, H), lambda i: (i, 0)),
            pl.BlockSpec((8, H), lambda i: (0, 0)),
            pl.BlockSpec((1, H), lambda i: (0, 0)),
            pl.BlockSpec((1, H), lambda i: (0, 0)),
            pl.BlockSpec((bs, H), lambda i: (i, 0)),
            pl.BlockSpec((H, H), lambda i: (0, 0)),
            pl.BlockSpec((H, H), lambda i: (0, 0)),
            pl.BlockSpec((1, H), lambda i: (0, 0)),
        ],
        out_specs=[
            pl.BlockSpec((bs, H), lambda i: (i, 0)),
            pl.BlockSpec((bs, H), lambda i: (i, 0)),
        ],
        out_shape=[
            jax.ShapeDtypeStruct((N_PAD, H), jnp.float32),
            jax.ShapeDtypeStruct((N_PAD, H), jnp.float32),
        ],
    )(hn, stats, gamma, beta, h, w_e1a, w_e1b, b_e1)


def _edge_body(u_ref, v_ref, w2_ref, b2_ref, w3_ref, o_ref):
    z1 = jnp.maximum(u_ref[...] + v_ref[...], 0.0)
    z2 = jnp.maximum(
        jnp.dot(z1, w2_ref[...], preferred_element_type=jnp.float32)
        + b2_ref[...], 0.0)
    o_ref[...] = jnp.dot(z2, w3_ref[...], preferred_element_type=jnp.float32)


def _edge_tc(u, v, w2, b2, w3):
    bs = 1024
    return pl.pallas_call(
        _edge_body,
        grid=(EB,),
        in_specs=[
            pl.BlockSpec((bs, H), lambda i: (i, 0)),
            pl.BlockSpec((bs, H), lambda i: (i, 0)),
            pl.BlockSpec((H, 32), lambda i: (0, 0)),
            pl.BlockSpec((1, 32), lambda i: (0, 0)),
            pl.BlockSpec((32, 1), lambda i: (0, 0)),
        ],
        out_specs=pl.BlockSpec((bs, 1), lambda i: (i, 0)),
        out_shape=jax.ShapeDtypeStruct((E_PAD, 1), jnp.float32),
    )(u, v, w2, b2, w3)


def kernel(x, edge_index, params):
    src = edge_index[0]
    dst = edge_index[1]
    src_p = jnp.concatenate(
        [src, jnp.zeros((E_PAD - E,), jnp.int32)]).reshape(E_PAD // ROW, ROW)
    dst_p = jnp.concatenate(
        [dst, jnp.full((E_PAD - E,), N, jnp.int32)]).reshape(E_PAD // ROW, ROW)
    x8 = jnp.pad(x, ((0, N_PAD - N), (0, 8 - F_IN)))

    srcq = [src_p * NCH + ch for ch in range(NCH)]
    degp = _deg_sc(dst_p)
    dinv = _dinv_tc(degp).reshape(N_PAD, 1)

    b_in = params['b_in'].reshape(1, H)
    h, hsf = _input_tc(
        x8, jnp.pad(params['W_in'], ((0, 8 - F_IN), (0, 0))), b_in,
        params['W_0'], dinv)

    for l in range(L):
        partials = _layer_sc(srcq[0], srcq[1], srcq[2], srcq[3], dst_p,
                             hsf.reshape(N_PAD * NCH, FC))
        hn, stats = _stats_tc(partials, hsf, dinv,
                              params[f'b_{l}'].reshape(1, H))
        gamma = params[f'gamma_{l}'].reshape(1, H)
        beta = params[f'beta_{l}'].reshape(1, H)
        if l < L - 1:
            h, hsf = _apply_tc(
                hn, stats, gamma, beta, h, dinv, params[f'W_{l + 1}'])
        else:
            a_tab, b_tab = _final_tc(
                hn, stats, gamma, beta, h,
                params['W_e1'][:H], params['W_e1'][H:],
                params['b_e1'].reshape(1, H))

    u, v = _edge_sc(a_tab, b_tab, src_p, dst_p)
    logits = _edge_tc(u, v, params['W_e2'], params['b_e2'].reshape(1, 32),
                      params['W_e3'])
    return (logits.reshape(E_PAD) + params['b_e3'][0])[:E]
